# baseline (speedup 1.0000x reference)
import functools
import jax
import jax.numpy as jnp
from jax.experimental import pallas as pl
from jax.experimental.pallas import tpu as pltpu

H = 128
NBASIS = 4


def _silu(v):
    return v * (1.0 / (1.0 + jnp.exp(-v)))


def _dense_e_body(x_ref, rbf_ref, jiW_ref, jib_ref, kjW_ref, kjb_ref,
                  rbfW_ref, xji_ref, xkj_ref):
    x = x_ref[...]
    xji_ref[...] = _silu(
        jnp.dot(x, jiW_ref[...], preferred_element_type=jnp.float32)
        + jib_ref[...])
    rbf_h = jnp.dot(rbf_ref[...], rbfW_ref[...],
                    preferred_element_type=jnp.float32)
    xkj_ref[...] = _silu(
        jnp.dot(x, kjW_ref[...], preferred_element_type=jnp.float32)
        + kjb_ref[...]) * rbf_h


def _dense_e(x, rbf, jiW, jib, kjW, kjb, rbfW, bE):
    E = x.shape[0]
    NR = rbf.shape[1]
    grid = (E // bE,)
    full = lambda shape: pl.BlockSpec(shape, lambda i: (0, 0))
    return pl.pallas_call(
        _dense_e_body,
        grid=grid,
        in_specs=[
            pl.BlockSpec((bE, H), lambda i: (i, 0)),
            pl.BlockSpec((bE, NR), lambda i: (i, 0)),
            full((H, H)), full((1, H)), full((H, H)), full((1, H)),
            full((NR, H)),
        ],
        out_specs=[
            pl.BlockSpec((bE, H), lambda i: (i, 0)),
            pl.BlockSpec((bE, H), lambda i: (i, 0)),
        ],
        out_shape=[
            jax.ShapeDtypeStruct((E, H), jnp.float32),
            jax.ShapeDtypeStruct((E, H), jnp.float32),
        ],
    )(x, rbf, jiW, jib, kjW, kjb, rbfW)


def _trip_body(sbf_ref, g_ref, sbfW_ref, Wm_ref, y_ref):
    sbf_h = jnp.dot(sbf_ref[...], sbfW_ref[...],
                    preferred_element_type=jnp.float32)
    g = g_ref[...]
    acc = jnp.zeros_like(y_ref)
    for j in range(NBASIS):
        t = jnp.dot(g, Wm_ref[j], preferred_element_type=jnp.float32)
        acc = acc + t * sbf_h[:, j:j + 1]
    y_ref[...] = acc


def _trip(sbf, g, sbfW, Wm, bT):
    T = g.shape[0]
    SR = sbf.shape[1]
    grid = (T // bT,)
    return pl.pallas_call(
        _trip_body,
        grid=grid,
        in_specs=[
            pl.BlockSpec((bT, SR), lambda i: (i, 0)),
            pl.BlockSpec((bT, H), lambda i: (i, 0)),
            pl.BlockSpec((SR, NBASIS), lambda i: (0, 0)),
            pl.BlockSpec((NBASIS, H, H), lambda i: (0, 0, 0)),
        ],
        out_specs=pl.BlockSpec((bT, H), lambda i: (i, 0)),
        out_shape=jax.ShapeDtypeStruct((T, H), jnp.float32),
    )(sbf, g, sbfW, Wm)


def _final_body(xji_ref, s_ref, W_ref, b_ref, out_ref):
    out_ref[...] = _silu(
        jnp.dot(xji_ref[...] + s_ref[...], W_ref[...],
                preferred_element_type=jnp.float32) + b_ref[...])


def _final(xji, s, W, b, bE):
    E = xji.shape[0]
    grid = (E // bE,)
    return pl.pallas_call(
        _final_body,
        grid=grid,
        in_specs=[
            pl.BlockSpec((bE, H), lambda i: (i, 0)),
            pl.BlockSpec((bE, H), lambda i: (i, 0)),
            pl.BlockSpec((H, H), lambda i: (0, 0)),
            pl.BlockSpec((1, H), lambda i: (0, 0)),
        ],
        out_specs=pl.BlockSpec((bE, H), lambda i: (i, 0)),
        out_shape=jax.ShapeDtypeStruct((E, H), jnp.float32),
    )(xji, s, W, b)


def kernel(x, rbf, sbf, idx_kj, idx_ji, lin_rbf_W, lin_sbf_W, lin_kj_W,
           lin_kj_b, lin_ji_W, lin_ji_b, W, lin_W, lin_b):
    E = x.shape[0]
    T = sbf.shape[0]
    bE = 3200 if E % 3200 == 0 else E
    bT = 5120 if T % 5120 == 0 else T

    x_ji, x_kj = _dense_e(x, rbf,
                          lin_ji_W, lin_ji_b.reshape(1, H),
                          lin_kj_W, lin_kj_b.reshape(1, H),
                          lin_rbf_W, bE)

    g = jnp.take(x_kj, idx_kj, axis=0)

    Wm = jnp.transpose(W, (1, 2, 0))
    y = _trip(sbf, g, lin_sbf_W, Wm, bT)

    s = jax.ops.segment_sum(y, idx_ji, num_segments=E)

    return _final(x_ji, s, lin_W, lin_b.reshape(1, H), bE)


# speedup vs baseline: 1.0779x; 1.0779x over previous
import functools
import jax
import jax.numpy as jnp
from jax import lax
from jax.experimental import pallas as pl
from jax.experimental.pallas import tpu as pltpu
from jax.experimental.pallas import tpu_sc as plsc

H = 128
NBASIS = 4


def _silu(v):
    return v * (1.0 / (1.0 + jnp.exp(-v)))


def _dense_e_body(x_ref, rbf_ref, jiW_ref, jib_ref, kjW_ref, kjb_ref,
                  rbfW_ref, xji_ref, xkj_ref):
    x = x_ref[...]
    xji_ref[...] = _silu(
        jnp.dot(x, jiW_ref[...], preferred_element_type=jnp.float32)
        + jib_ref[...])
    rbf_h = jnp.dot(rbf_ref[...], rbfW_ref[...],
                    preferred_element_type=jnp.float32)
    xkj_ref[...] = _silu(
        jnp.dot(x, kjW_ref[...], preferred_element_type=jnp.float32)
        + kjb_ref[...]) * rbf_h


def _dense_e(x, rbf, jiW, jib, kjW, kjb, rbfW, bE):
    E = x.shape[0]
    NR = rbf.shape[1]
    grid = (E // bE,)
    full = lambda shape: pl.BlockSpec(shape, lambda i: (0, 0))
    return pl.pallas_call(
        _dense_e_body,
        grid=grid,
        in_specs=[
            pl.BlockSpec((bE, H), lambda i: (i, 0)),
            pl.BlockSpec((bE, NR), lambda i: (i, 0)),
            full((H, H)), full((1, H)), full((H, H)), full((1, H)),
            full((NR, H)),
        ],
        out_specs=[
            pl.BlockSpec((bE, H), lambda i: (i, 0)),
            pl.BlockSpec((bE, H), lambda i: (i, 0)),
        ],
        out_shape=[
            jax.ShapeDtypeStruct((E, H), jnp.float32),
            jax.ShapeDtypeStruct((E, H), jnp.float32),
        ],
    )(x, rbf, jiW, jib, kjW, kjb, rbfW)


def _trip_body(sbf_ref, g_ref, sbfW_ref, Wm_ref, y_ref):
    sbf_h = jnp.dot(sbf_ref[...], sbfW_ref[...],
                    preferred_element_type=jnp.float32)
    g = g_ref[...]
    acc = jnp.zeros_like(y_ref)
    for j in range(NBASIS):
        t = jnp.dot(g, Wm_ref[j], preferred_element_type=jnp.float32)
        acc = acc + t * sbf_h[:, j:j + 1]
    y_ref[...] = acc


def _trip(sbf, g, sbfW, Wm, bT):
    T = g.shape[0]
    SR = sbf.shape[1]
    grid = (T // bT,)
    return pl.pallas_call(
        _trip_body,
        grid=grid,
        in_specs=[
            pl.BlockSpec((bT, SR), lambda i: (i, 0)),
            pl.BlockSpec((bT, H), lambda i: (i, 0)),
            pl.BlockSpec((SR, NBASIS), lambda i: (0, 0)),
            pl.BlockSpec((NBASIS, H, H), lambda i: (0, 0, 0)),
        ],
        out_specs=pl.BlockSpec((bT, H), lambda i: (i, 0)),
        out_shape=jax.ShapeDtypeStruct((T, H), jnp.float32),
    )(sbf, g, sbfW, Wm)


def _final_body(xji_ref, s_ref, W_ref, b_ref, out_ref):
    out_ref[...] = _silu(
        jnp.dot(xji_ref[...] + s_ref[...], W_ref[...],
                preferred_element_type=jnp.float32) + b_ref[...])


def _final(xji, s, W, b, bE):
    E = xji.shape[0]
    grid = (E // bE,)
    return pl.pallas_call(
        _final_body,
        grid=grid,
        in_specs=[
            pl.BlockSpec((bE, H), lambda i: (i, 0)),
            pl.BlockSpec((bE, H), lambda i: (i, 0)),
            pl.BlockSpec((H, H), lambda i: (0, 0)),
            pl.BlockSpec((1, H), lambda i: (0, 0)),
        ],
        out_specs=pl.BlockSpec((bE, H), lambda i: (i, 0)),
        out_shape=jax.ShapeDtypeStruct((E, H), jnp.float32),
    )(xji, s, W, b)


SC_NC = 2
SC_NT = 16
CH = 8192
SB = 128


def _scatter_make(T, E):
    nchg = (E + CH - 1) // CH
    ncs = nchg // SC_NC
    mesh = plsc.VectorSubcoreMesh(core_axis_name="c", subcore_axis_name="s")

    @functools.partial(
        pl.kernel, mesh=mesh,
        out_type=jax.ShapeDtypeStruct((E, H), jnp.float32),
        scratch_types=[
            pltpu.VMEM((64,), jnp.int32),
            pltpu.VMEM((SB,), jnp.int32),
            pltpu.VMEM((SB,), jnp.int32),
            pltpu.VMEM((SB,), jnp.int32),
            pltpu.VMEM((SB, H), jnp.float32),
            pltpu.VMEM_SHARED((CH + 8, H), jnp.float32),
        ],
    )
    def scat(y_hbm, perm_hbm, idxs_hbm, bnd_hbm, out_hbm,
             bounds, wbuf, ibuf, dbuf, ybuf, acc):
        c = lax.axis_index("c")
        s = lax.axis_index("s")
        lane = lax.iota(jnp.int32, 16)
        pltpu.sync_copy(bnd_hbm, bounds)

        def chunk_body(j, _):
            ch = c * ncs + j
            lo = ch * CH
            rows = jnp.minimum(E - lo, CH)
            npiece = rows // SB

            def zfill(i, _):
                r = i // (H // 16)
                col = (i % (H // 16)) * 16
                ybuf[r, pl.ds(col, 16)] = jnp.zeros((16,), jnp.float32)
                return 0
            lax.fori_loop(0, SB * (H // 16), zfill, 0)
            for p in range(CH // SB // SC_NT):
                q = p * SC_NT + s

                @pl.when(q < npiece)
                def _():
                    pltpu.sync_copy(ybuf, acc.at[pl.ds(q * SB, SB)])
            plsc.subcore_barrier()

            bv = bounds[pl.ds(ch, 16)]
            n0 = bv[0]
            n1 = bv[1]
            share = n1 - n0
            t_s = n0 + (share * s) // SC_NT
            t_e = n0 + (share * (s + 1)) // SC_NT
            base0 = t_s - (t_s & 7)

            def batch(k, _):
                base = pl.multiple_of(base0 + k * SB, 8)
                pltpu.sync_copy(perm_hbm.at[pl.ds(base, SB)], wbuf)
                pltpu.sync_copy(idxs_hbm.at[pl.ds(base, SB)], ibuf)
                for u in range(SB // 16):
                    pos = base + u * 16 + lane
                    idxv = ibuf[pl.ds(u * 16, 16)]
                    ok = (pos >= t_s) & (pos < t_e)
                    dv = jnp.where(ok, idxv - lo, CH + (lane & 7))
                    dbuf[pl.ds(u * 16, 16)] = dv
                pltpu.sync_copy(y_hbm.at[wbuf], ybuf)
                pltpu.sync_copy(ybuf, acc.at[dbuf], add=True)
                return 0
            lax.fori_loop(0, (t_e - base0 + SB - 1) // SB, batch, 0)
            plsc.subcore_barrier()

            for p in range(CH // SB // SC_NT):
                q = p * SC_NT + s

                @pl.when(q < npiece)
                def _():
                    pltpu.sync_copy(acc.at[pl.ds(q * SB, SB)], ybuf)
                    pltpu.sync_copy(ybuf, out_hbm.at[pl.ds(lo + q * SB, SB)])
            return 0

        lax.fori_loop(0, ncs, chunk_body, 0)

    return scat


def _scatter(y, idx_ji, E):
    T = y.shape[0]
    nchg = (E + CH - 1) // CH
    bucket = idx_ji // CH
    perm = jnp.argsort(bucket).astype(jnp.int32)
    idx_sorted = jnp.take(idx_ji, perm)
    bsorted = jnp.take(bucket, perm)
    bnd = jnp.searchsorted(bsorted, jnp.arange(nchg, dtype=idx_ji.dtype)
                           ).astype(jnp.int32)
    bnd = jnp.concatenate([bnd, jnp.full((64 - nchg,), T, jnp.int32)])
    pad = jnp.zeros((2 * SB,), jnp.int32)
    perm_p = jnp.concatenate([perm, pad])
    idxs_p = jnp.concatenate([idx_sorted, pad])
    return _scatter_make(T, E)(y, perm_p, idxs_p, bnd)


def kernel(x, rbf, sbf, idx_kj, idx_ji, lin_rbf_W, lin_sbf_W, lin_kj_W,
           lin_kj_b, lin_ji_W, lin_ji_b, W, lin_W, lin_b):
    E = x.shape[0]
    T = sbf.shape[0]
    bE = 3200 if E % 3200 == 0 else E
    bT = 5120 if T % 5120 == 0 else T

    x_ji, x_kj = _dense_e(x, rbf,
                          lin_ji_W, lin_ji_b.reshape(1, H),
                          lin_kj_W, lin_kj_b.reshape(1, H),
                          lin_rbf_W, bE)

    g = jnp.take(x_kj, idx_kj, axis=0)

    Wm = jnp.transpose(W, (1, 2, 0))
    y = _trip(sbf, g, lin_sbf_W, Wm, bT)

    if E % SB == 0 and ((E + CH - 1) // CH) % SC_NC == 0:
        s = _scatter(y, idx_ji, E)
    else:
        s = jax.ops.segment_sum(y, idx_ji, num_segments=E)

    return _final(x_ji, s, lin_W, lin_b.reshape(1, H), bE)


# speedup vs baseline: 1.1377x; 1.0554x over previous
import functools
import jax
import jax.numpy as jnp
from jax import lax
from jax.experimental import pallas as pl
from jax.experimental.pallas import tpu as pltpu
from jax.experimental.pallas import tpu_sc as plsc

H = 128
NBASIS = 4


def _silu(v):
    return v * (1.0 / (1.0 + jnp.exp(-v)))


def _dense_e_body(x_ref, rbf_ref, jiW_ref, jib_ref, kjW_ref, kjb_ref,
                  rbfW_ref, xji_ref, xkj_ref):
    x = x_ref[...]
    xji_ref[...] = _silu(
        jnp.dot(x, jiW_ref[...], preferred_element_type=jnp.float32)
        + jib_ref[...])
    rbf_h = jnp.dot(rbf_ref[...], rbfW_ref[...],
                    preferred_element_type=jnp.float32)
    xkj_ref[...] = _silu(
        jnp.dot(x, kjW_ref[...], preferred_element_type=jnp.float32)
        + kjb_ref[...]) * rbf_h


def _dense_e(x, rbf, jiW, jib, kjW, kjb, rbfW, bE):
    E = x.shape[0]
    NR = rbf.shape[1]
    grid = (E // bE,)
    full = lambda shape: pl.BlockSpec(shape, lambda i: (0, 0))
    return pl.pallas_call(
        _dense_e_body,
        grid=grid,
        in_specs=[
            pl.BlockSpec((bE, H), lambda i: (i, 0)),
            pl.BlockSpec((bE, NR), lambda i: (i, 0)),
            full((H, H)), full((1, H)), full((H, H)), full((1, H)),
            full((NR, H)),
        ],
        out_specs=[
            pl.BlockSpec((bE, H), lambda i: (i, 0)),
            pl.BlockSpec((bE, H), lambda i: (i, 0)),
        ],
        out_shape=[
            jax.ShapeDtypeStruct((E, H), jnp.float32),
            jax.ShapeDtypeStruct((E, H), jnp.float32),
        ],
    )(x, rbf, jiW, jib, kjW, kjb, rbfW)


def _trip_body(sbf_ref, g_ref, sbfW_ref, Wm_ref, y_ref):
    sbf_h = jnp.dot(sbf_ref[...], sbfW_ref[...],
                    preferred_element_type=jnp.float32)
    g = g_ref[...].astype(jnp.bfloat16)
    acc = jnp.zeros_like(y_ref)
    for j in range(NBASIS):
        t = jnp.dot(g, Wm_ref[j], preferred_element_type=jnp.float32)
        acc = acc + t * sbf_h[:, j:j + 1]
    y_ref[...] = acc


def _trip(sbf, g, sbfW, Wm, bT):
    T = g.shape[0]
    SR = sbf.shape[1]
    grid = (T // bT,)
    return pl.pallas_call(
        _trip_body,
        grid=grid,
        in_specs=[
            pl.BlockSpec((bT, SR), lambda i: (i, 0)),
            pl.BlockSpec((bT, H), lambda i: (i, 0)),
            pl.BlockSpec((SR, NBASIS), lambda i: (0, 0)),
            pl.BlockSpec((NBASIS, H, H), lambda i: (0, 0, 0)),
        ],
        out_specs=pl.BlockSpec((bT, H), lambda i: (i, 0)),
        out_shape=jax.ShapeDtypeStruct((T, H), jnp.float32),
    )(sbf, g, sbfW, Wm)


def _final_body(xji_ref, s_ref, W_ref, b_ref, out_ref):
    out_ref[...] = _silu(
        jnp.dot(xji_ref[...] + s_ref[...], W_ref[...],
                preferred_element_type=jnp.float32) + b_ref[...])


def _final(xji, s, W, b, bE):
    E = xji.shape[0]
    grid = (E // bE,)
    return pl.pallas_call(
        _final_body,
        grid=grid,
        in_specs=[
            pl.BlockSpec((bE, H), lambda i: (i, 0)),
            pl.BlockSpec((bE, H), lambda i: (i, 0)),
            pl.BlockSpec((H, H), lambda i: (0, 0)),
            pl.BlockSpec((1, H), lambda i: (0, 0)),
        ],
        out_specs=pl.BlockSpec((bE, H), lambda i: (i, 0)),
        out_shape=jax.ShapeDtypeStruct((E, H), jnp.float32),
    )(xji, s, W, b)


SC_NC = 2
SC_NT = 16
CH = 8192
SB = 128


def _scatter_make(T, E):
    nchg = (E + CH - 1) // CH
    ncs = nchg // SC_NC
    mesh = plsc.VectorSubcoreMesh(core_axis_name="c", subcore_axis_name="s")

    @functools.partial(
        pl.kernel, mesh=mesh,
        out_type=jax.ShapeDtypeStruct((E, H), jnp.float32),
        scratch_types=[
            pltpu.VMEM((64,), jnp.int32),
            pltpu.VMEM((SB,), jnp.int32),
            pltpu.VMEM((SB,), jnp.int32),
            pltpu.VMEM((SB,), jnp.int32),
            pltpu.VMEM((SB, H), jnp.float32),
            pltpu.VMEM((SB,), jnp.int32),
            pltpu.VMEM((SB,), jnp.int32),
            pltpu.VMEM((SB,), jnp.int32),
            pltpu.VMEM((SB, H), jnp.float32),
            pltpu.SemaphoreType.DMA,
            pltpu.SemaphoreType.DMA,
            pltpu.VMEM_SHARED((CH + 8, H), jnp.float32),
        ],
    )
    def scat(y_hbm, perm_hbm, idxs_hbm, bnd_hbm, out_hbm,
             bounds, wbuf, ibuf, dbuf, ybuf,
             wbuf2, ibuf2, dbuf2, ybuf2, semA, semB, acc):
        c = lax.axis_index("c")
        s = lax.axis_index("s")
        lane = lax.iota(jnp.int32, 16)
        pltpu.sync_copy(bnd_hbm, bounds)

        def chunk_body(j, _):
            ch = c * ncs + j
            lo = ch * CH
            rows = jnp.minimum(E - lo, CH)
            npiece = rows // SB

            def zfill(i, _):
                r = i // (H // 16)
                col = (i % (H // 16)) * 16
                ybuf[r, pl.ds(col, 16)] = jnp.zeros((16,), jnp.float32)
                return 0
            lax.fori_loop(0, SB * (H // 16), zfill, 0)
            for p in range(CH // SB // SC_NT):
                q = p * SC_NT + s

                @pl.when(q < npiece)
                def _():
                    pltpu.sync_copy(ybuf, acc.at[pl.ds(q * SB, SB)])
            plsc.subcore_barrier()

            bv = bounds[pl.ds(ch, 16)]
            n0 = bv[0]
            n1 = bv[1]
            share = n1 - n0
            t_s = n0 + (share * s) // SC_NT
            t_e = n0 + (share * (s + 1)) // SC_NT
            base0 = t_s - (t_s & 7)

            nb = (t_e - base0 + SB - 1) // SB

            def prep(k, wb, ib, db, yb, sem):
                base = pl.multiple_of(base0 + k * SB, 8)
                pltpu.sync_copy(perm_hbm.at[pl.ds(base, SB)], wb)
                pltpu.sync_copy(idxs_hbm.at[pl.ds(base, SB)], ib)
                for u in range(SB // 16):
                    pos = base + u * 16 + lane
                    idxv = ib[pl.ds(u * 16, 16)]
                    ok = (pos >= t_s) & (pos < t_e)
                    dv = jnp.where(ok, idxv - lo, CH + (lane & 7))
                    db[pl.ds(u * 16, 16)] = dv
                pltpu.make_async_copy(y_hbm.at[wb], yb, sem).start()

            @pl.when(nb > 0)
            def _():
                prep(0, wbuf, ibuf, dbuf, ybuf, semA)

            def pair(i, _):
                b0 = 2 * i
                b1 = b0 + 1

                @pl.when(b1 < nb)
                def _():
                    prep(b1, wbuf2, ibuf2, dbuf2, ybuf2, semB)
                pltpu.make_async_copy(y_hbm.at[wbuf], ybuf, semA).wait()
                pltpu.sync_copy(ybuf, acc.at[dbuf], add=True)

                @pl.when(b0 + 2 < nb)
                def _():
                    prep(b0 + 2, wbuf, ibuf, dbuf, ybuf, semA)

                @pl.when(b1 < nb)
                def _():
                    pltpu.make_async_copy(y_hbm.at[wbuf2], ybuf2, semB).wait()
                    pltpu.sync_copy(ybuf2, acc.at[dbuf2], add=True)
                return 0
            lax.fori_loop(0, (nb + 1) // 2, pair, 0)
            plsc.subcore_barrier()

            for p in range(CH // SB // SC_NT):
                q = p * SC_NT + s

                @pl.when(q < npiece)
                def _():
                    pltpu.sync_copy(acc.at[pl.ds(q * SB, SB)], ybuf)
                    pltpu.sync_copy(ybuf, out_hbm.at[pl.ds(lo + q * SB, SB)])
            return 0

        lax.fori_loop(0, ncs, chunk_body, 0)

    return scat


def _scatter(y, idx_ji, E):
    T = y.shape[0]
    nchg = (E + CH - 1) // CH
    bucket = idx_ji // CH
    perm = jnp.argsort(bucket).astype(jnp.int32)
    idx_sorted = jnp.take(idx_ji, perm)
    bsorted = jnp.take(bucket, perm)
    bnd = jnp.searchsorted(bsorted, jnp.arange(nchg, dtype=idx_ji.dtype)
                           ).astype(jnp.int32)
    bnd = jnp.concatenate([bnd, jnp.full((64 - nchg,), T, jnp.int32)])
    pad = jnp.zeros((2 * SB,), jnp.int32)
    perm_p = jnp.concatenate([perm, pad])
    idxs_p = jnp.concatenate([idx_sorted, pad])
    return _scatter_make(T, E)(y, perm_p, idxs_p, bnd)


def kernel(x, rbf, sbf, idx_kj, idx_ji, lin_rbf_W, lin_sbf_W, lin_kj_W,
           lin_kj_b, lin_ji_W, lin_ji_b, W, lin_W, lin_b):
    E = x.shape[0]
    T = sbf.shape[0]
    bE = 3200 if E % 3200 == 0 else E
    bT = 5120 if T % 5120 == 0 else T

    x_ji, x_kj = _dense_e(x, rbf,
                          lin_ji_W, lin_ji_b.reshape(1, H),
                          lin_kj_W, lin_kj_b.reshape(1, H),
                          lin_rbf_W, bE)

    g = jnp.take(x_kj, idx_kj, axis=0)

    Wm = jnp.transpose(W, (1, 2, 0)).astype(jnp.bfloat16)
    y = _trip(sbf, g, lin_sbf_W, Wm, bT)

    if E % SB == 0 and ((E + CH - 1) // CH) % SC_NC == 0:
        s = _scatter(y, idx_ji, E)
    else:
        s = jax.ops.segment_sum(y, idx_ji, num_segments=E)

    return _final(x_ji, s, lin_W, lin_b.reshape(1, H), bE)


# speedup vs baseline: 1.8959x; 1.6665x over previous
import functools
import jax
import jax.numpy as jnp
from jax import lax
from jax.experimental import pallas as pl
from jax.experimental.pallas import tpu as pltpu
from jax.experimental.pallas import tpu_sc as plsc

H = 128
NBASIS = 4


def _silu(v):
    return v * (1.0 / (1.0 + jnp.exp(-v)))


def _dense_e_body(x_ref, rbf_ref, jiW_ref, jib_ref, kjW_ref, kjb_ref,
                  rbfW_ref, xji_ref, xkj_ref):
    x = x_ref[...]
    xji_ref[...] = _silu(
        jnp.dot(x, jiW_ref[...], preferred_element_type=jnp.float32)
        + jib_ref[...])
    rbf_h = jnp.dot(rbf_ref[...], rbfW_ref[...],
                    preferred_element_type=jnp.float32)
    xkj_ref[...] = _silu(
        jnp.dot(x, kjW_ref[...], preferred_element_type=jnp.float32)
        + kjb_ref[...]) * rbf_h


def _dense_e(x, rbf, jiW, jib, kjW, kjb, rbfW, bE):
    E = x.shape[0]
    NR = rbf.shape[1]
    grid = (E // bE,)
    full = lambda shape: pl.BlockSpec(shape, lambda i: (0, 0))
    return pl.pallas_call(
        _dense_e_body,
        grid=grid,
        in_specs=[
            pl.BlockSpec((bE, H), lambda i: (i, 0)),
            pl.BlockSpec((bE, NR), lambda i: (i, 0)),
            full((H, H)), full((1, H)), full((H, H)), full((1, H)),
            full((NR, H)),
        ],
        out_specs=[
            pl.BlockSpec((bE, H), lambda i: (i, 0)),
            pl.BlockSpec((bE, H), lambda i: (i, 0)),
        ],
        out_shape=[
            jax.ShapeDtypeStruct((E, H), jnp.float32),
            jax.ShapeDtypeStruct((E, H), jnp.float32),
        ],
    )(x, rbf, jiW, jib, kjW, kjb, rbfW)


def _trip_body(sbf_ref, g_ref, sbfW_ref, Wm_ref, y_ref):
    sbf_h = jnp.dot(sbf_ref[...], sbfW_ref[...],
                    preferred_element_type=jnp.float32)
    g = g_ref[...].astype(jnp.bfloat16)
    acc = jnp.zeros_like(y_ref)
    for j in range(NBASIS):
        t = jnp.dot(g, Wm_ref[j], preferred_element_type=jnp.float32)
        acc = acc + t * sbf_h[:, j:j + 1]
    y_ref[...] = acc


def _trip(sbf, g, sbfW, Wm, bT):
    T = g.shape[0]
    SR = sbf.shape[1]
    grid = (T // bT,)
    return pl.pallas_call(
        _trip_body,
        grid=grid,
        in_specs=[
            pl.BlockSpec((bT, SR), lambda i: (i, 0)),
            pl.BlockSpec((bT, H), lambda i: (i, 0)),
            pl.BlockSpec((SR, NBASIS), lambda i: (0, 0)),
            pl.BlockSpec((NBASIS, H, H), lambda i: (0, 0, 0)),
        ],
        out_specs=pl.BlockSpec((bT, H), lambda i: (i, 0)),
        out_shape=jax.ShapeDtypeStruct((T, H), jnp.float32),
    )(sbf, g, sbfW, Wm)


def _final_body(xji_ref, s_ref, W_ref, b_ref, out_ref):
    out_ref[...] = _silu(
        jnp.dot(xji_ref[...] + s_ref[...], W_ref[...],
                preferred_element_type=jnp.float32) + b_ref[...])


def _final(xji, s, W, b, bE):
    E = xji.shape[0]
    grid = (E // bE,)
    return pl.pallas_call(
        _final_body,
        grid=grid,
        in_specs=[
            pl.BlockSpec((bE, H), lambda i: (i, 0)),
            pl.BlockSpec((bE, H), lambda i: (i, 0)),
            pl.BlockSpec((H, H), lambda i: (0, 0)),
            pl.BlockSpec((1, H), lambda i: (0, 0)),
        ],
        out_specs=pl.BlockSpec((bE, H), lambda i: (i, 0)),
        out_shape=jax.ShapeDtypeStruct((E, H), jnp.float32),
    )(xji, s, W, b)


SC_NC = 2
SC_NT = 16
CH = 8192
SB = 128


def _scatter_make(T, E):
    nchg = (E + CH - 1) // CH
    ncs = nchg // SC_NC
    mesh = plsc.VectorSubcoreMesh(core_axis_name="c", subcore_axis_name="s")

    @functools.partial(
        pl.kernel, mesh=mesh,
        out_type=jax.ShapeDtypeStruct((E, H), jnp.float32),
        scratch_types=[
            pltpu.VMEM((64,), jnp.int32),
            pltpu.VMEM((SB,), jnp.int32),
            pltpu.VMEM((SB,), jnp.int32),
            pltpu.VMEM((SB,), jnp.int32),
            pltpu.VMEM((SB, H), jnp.float32),
            pltpu.VMEM((SB,), jnp.int32),
            pltpu.VMEM((SB,), jnp.int32),
            pltpu.VMEM((SB,), jnp.int32),
            pltpu.VMEM((SB, H), jnp.float32),
            pltpu.SemaphoreType.DMA,
            pltpu.SemaphoreType.DMA,
            pltpu.VMEM_SHARED((CH + 8, H), jnp.float32),
        ],
    )
    def scat(y_hbm, perm_hbm, idxs_hbm, bnd_hbm, out_hbm,
             bounds, wbuf, ibuf, dbuf, ybuf,
             wbuf2, ibuf2, dbuf2, ybuf2, semA, semB, acc):
        c = lax.axis_index("c")
        s = lax.axis_index("s")
        lane = lax.iota(jnp.int32, 16)
        pltpu.sync_copy(bnd_hbm, bounds)

        def chunk_body(j, _):
            ch = c * ncs + j
            lo = ch * CH
            rows = jnp.minimum(E - lo, CH)
            npiece = rows // SB

            def zfill(i, _):
                r = i // (H // 16)
                col = (i % (H // 16)) * 16
                ybuf[r, pl.ds(col, 16)] = jnp.zeros((16,), jnp.float32)
                return 0
            lax.fori_loop(0, SB * (H // 16), zfill, 0)
            for p in range(CH // SB // SC_NT):
                q = p * SC_NT + s

                @pl.when(q < npiece)
                def _():
                    pltpu.sync_copy(ybuf, acc.at[pl.ds(q * SB, SB)])
            plsc.subcore_barrier()

            bv = bounds[pl.ds(ch, 16)]
            n0 = bv[0]
            n1 = bv[1]
            share = n1 - n0
            t_s = n0 + (share * s) // SC_NT
            t_e = n0 + (share * (s + 1)) // SC_NT
            base0 = t_s - (t_s & 7)

            nb = (t_e - base0 + SB - 1) // SB

            def prep(k, wb, ib, db, yb, sem):
                base = pl.multiple_of(base0 + k * SB, 8)
                pltpu.sync_copy(perm_hbm.at[pl.ds(base, SB)], wb)
                pltpu.sync_copy(idxs_hbm.at[pl.ds(base, SB)], ib)
                for u in range(SB // 16):
                    pos = base + u * 16 + lane
                    idxv = ib[pl.ds(u * 16, 16)]
                    ok = (pos >= t_s) & (pos < t_e)
                    dv = jnp.where(ok, idxv - lo, CH + (lane & 7))
                    db[pl.ds(u * 16, 16)] = dv
                pltpu.make_async_copy(y_hbm.at[wb], yb, sem).start()

            @pl.when(nb > 0)
            def _():
                prep(0, wbuf, ibuf, dbuf, ybuf, semA)

            def pair(i, _):
                b0 = 2 * i
                b1 = b0 + 1

                @pl.when(b1 < nb)
                def _():
                    prep(b1, wbuf2, ibuf2, dbuf2, ybuf2, semB)
                pltpu.make_async_copy(y_hbm.at[wbuf], ybuf, semA).wait()
                pltpu.sync_copy(ybuf, acc.at[dbuf], add=True)

                @pl.when(b0 + 2 < nb)
                def _():
                    prep(b0 + 2, wbuf, ibuf, dbuf, ybuf, semA)

                @pl.when(b1 < nb)
                def _():
                    pltpu.make_async_copy(y_hbm.at[wbuf2], ybuf2, semB).wait()
                    pltpu.sync_copy(ybuf2, acc.at[dbuf2], add=True)
                return 0
            lax.fori_loop(0, (nb + 1) // 2, pair, 0)
            plsc.subcore_barrier()

            for p in range(CH // SB // SC_NT):
                q = p * SC_NT + s

                @pl.when(q < npiece)
                def _():
                    pltpu.sync_copy(acc.at[pl.ds(q * SB, SB)], ybuf)
                    pltpu.sync_copy(ybuf, out_hbm.at[pl.ds(lo + q * SB, SB)])
            return 0

        lax.fori_loop(0, ncs, chunk_body, 0)

    return scat


def _scatter(y, idx_ji, E):
    T = y.shape[0]
    nchg = (E + CH - 1) // CH
    bucket = idx_ji // CH
    perm = jnp.argsort(bucket).astype(jnp.int32)
    idx_sorted = jnp.take(idx_ji, perm)
    bsorted = jnp.take(bucket, perm)
    bnd = jnp.searchsorted(bsorted, jnp.arange(nchg, dtype=idx_ji.dtype)
                           ).astype(jnp.int32)
    bnd = jnp.concatenate([bnd, jnp.full((64 - nchg,), T, jnp.int32)])
    pad = jnp.zeros((2 * SB,), jnp.int32)
    perm_p = jnp.concatenate([perm, pad])
    idxs_p = jnp.concatenate([idx_sorted, pad])
    return _scatter_make(T, E)(y, perm_p, idxs_p, bnd)


GB = 128


def _gather_make(T, E):
    nt = SC_NC * SC_NT
    share = T // nt
    nfull = share // GB
    rem = share - nfull * GB
    mesh = plsc.VectorSubcoreMesh(core_axis_name="c", subcore_axis_name="s")

    @functools.partial(
        pl.kernel, mesh=mesh,
        out_type=jax.ShapeDtypeStruct((T, H), jnp.float32),
        scratch_types=[
            pltpu.VMEM((GB,), jnp.int32),
            pltpu.VMEM((GB, H), jnp.float32),
            pltpu.VMEM((GB,), jnp.int32),
            pltpu.VMEM((GB, H), jnp.float32),
            pltpu.SemaphoreType.DMA,
            pltpu.SemaphoreType.DMA,
        ],
    )
    def gat(tab_hbm, idx_hbm, out_hbm, wA, yA, wB, yB, sA, sB):
        c = lax.axis_index("c")
        s = lax.axis_index("s")
        tid = s * SC_NC + c
        base = tid * share

        def prep(k, wb, yb, sem):
            pltpu.sync_copy(idx_hbm.at[pl.ds(base + k * GB, GB)], wb)
            pltpu.make_async_copy(tab_hbm.at[wb], yb, sem).start()

        prep(0, wA, yA, sA)

        def pair(i, _):
            b0 = 2 * i
            b1 = b0 + 1

            @pl.when(b1 < nfull)
            def _():
                prep(b1, wB, yB, sB)
            pltpu.make_async_copy(tab_hbm.at[wA], yA, sA).wait()
            pltpu.sync_copy(yA, out_hbm.at[pl.ds(base + b0 * GB, GB)])

            @pl.when(b0 + 2 < nfull)
            def _():
                prep(b0 + 2, wA, yA, sA)

            @pl.when(b1 < nfull)
            def _():
                pltpu.make_async_copy(tab_hbm.at[wB], yB, sB).wait()
                pltpu.sync_copy(yB, out_hbm.at[pl.ds(base + b1 * GB, GB)])
            return 0
        lax.fori_loop(0, (nfull + 1) // 2, pair, 0)
        if rem:
            tb = base + nfull * GB
            pltpu.sync_copy(idx_hbm.at[pl.ds(tb, rem)], wA.at[pl.ds(0, rem)])
            pltpu.sync_copy(tab_hbm.at[wA.at[pl.ds(0, rem)]],
                            yA.at[pl.ds(0, rem)])
            pltpu.sync_copy(yA.at[pl.ds(0, rem)], out_hbm.at[pl.ds(tb, rem)])

    return gat


def kernel(x, rbf, sbf, idx_kj, idx_ji, lin_rbf_W, lin_sbf_W, lin_kj_W,
           lin_kj_b, lin_ji_W, lin_ji_b, W, lin_W, lin_b):
    E = x.shape[0]
    T = sbf.shape[0]
    bE = 3200 if E % 3200 == 0 else E
    bT = 5120 if T % 5120 == 0 else T

    x_ji, x_kj = _dense_e(x, rbf,
                          lin_ji_W, lin_ji_b.reshape(1, H),
                          lin_kj_W, lin_kj_b.reshape(1, H),
                          lin_rbf_W, bE)

    nt = SC_NC * SC_NT
    if T % nt == 0 and (T // nt) % 8 == 0:
        g = _gather_make(T, E)(x_kj, idx_kj)
    else:
        g = jnp.take(x_kj, idx_kj, axis=0)

    Wm = jnp.transpose(W, (1, 2, 0)).astype(jnp.bfloat16)
    y = _trip(sbf, g, lin_sbf_W, Wm, bT)

    if E % SB == 0 and ((E + CH - 1) // CH) % SC_NC == 0:
        s = _scatter(y, idx_ji, E)
    else:
        s = jax.ops.segment_sum(y, idx_ji, num_segments=E)

    return _final(x_ji, s, lin_W, lin_b.reshape(1, H), bE)
